# baseline (speedup 1.0000x reference)
import functools
import math

import jax
import jax.numpy as jnp
import numpy as np
from jax import lax
from jax.experimental import pallas as pl
from jax.experimental.pallas import tpu as pltpu
from jax.experimental.pallas import tpu_sc as plsc

_D_MODEL = 128
_N_BINS = 16
_LANES = 16
_N_WORKERS = 32
_CPS_PAD = 32


def _pe_table_t(d_model, length):
    position = np.arange(length, dtype=np.float32)[:, None]
    div_term = np.exp(np.arange(0, d_model, 2, dtype=np.float32)
                      * -(math.log(10000.0) / d_model))
    pe = np.zeros((length, d_model), dtype=np.float32)
    pe[:, 0::2] = np.sin(position * div_term)
    pe[:, 1::2] = np.cos(position * div_term)
    return pe.T.copy()


def _tc_body(x_ref, pebuf_t_ref, wpos_ref, xmark_t_ref, wte_ref,
             bpos_ref, bte_ref, cps_ref, q_ref, ei_ref, ef_ref):
    pe = jnp.sum(pebuf_t_ref[:] * wpos_ref[:], axis=0, keepdims=True)
    te = jnp.sum(xmark_t_ref[:] * wte_ref[:], axis=0, keepdims=True)
    bias = bpos_ref[0] + bte_ref[0]
    xcols = jnp.transpose(x_ref[:], (0, 2, 1)).reshape(q_ref.shape)
    u = xcols + pe + te + bias
    length = u.shape[1]
    k = 1
    while k < length:
        shifted = jnp.concatenate(
            [jnp.zeros(u.shape[:-1] + (k,), u.dtype), u[:, :length - k]],
            axis=1)
        u = u + shifted
        k *= 2
    q_ref[:] = u
    cps = cps_ref[:]
    lens = cps[:, 1:] - cps[:, :-1]
    m = jnp.max(lens)
    i = lax.broadcasted_iota(jnp.int32, (1, _N_BINS), 1)
    lo = jnp.right_shift(i * m, 4)
    hi = jnp.right_shift((i + 1) * m + 15, 4)
    ei_ref[:] = jnp.reshape(
        jnp.concatenate(
            [lo, hi, jnp.zeros((1, 128 - 2 * _N_BINS), jnp.int32)], axis=1),
        (128,))
    inv = 1.0 / (hi - lo).astype(jnp.float32)
    ef_ref[:] = jnp.reshape(
        jnp.concatenate(
            [inv, jnp.zeros((1, 128 - _N_BINS), jnp.float32)], axis=1),
        (128,))


def _sc_body(n_rows, n_seg, q_hbm, cps_hbm, ei_hbm, ef_hbm, out_hbm,
             cpsall_v, ei_v, ef_v, q0_v, q1_v, o0_v, o1_v, sem):
    wid = lax.axis_index("c") * 16 + lax.axis_index("s")
    n0 = wid
    n1 = wid + _N_WORKERS
    has2 = n1 < n_rows
    n1c = jnp.where(has2, n1, 0)
    copies = [
        pltpu.async_copy(cps_hbm, cpsall_v, sem),
        pltpu.async_copy(ei_hbm, ei_v, sem),
        pltpu.async_copy(ef_hbm, ef_v, sem),
        pltpu.async_copy(q_hbm.at[n0], q0_v, sem),
        pltpu.async_copy(q_hbm.at[n1c], q1_v, sem),
    ]
    for c in copies:
        c.wait()
    lo_vec = ei_v[pl.ds(0, _LANES)]
    hi_vec = ei_v[pl.ds(_LANES, _LANES)]
    inv_vec = ef_v[pl.ds(0, _LANES)]

    def do_row(n, qrow_v, orow_v):
        base = jnp.full((_LANES,), n * _CPS_PAD, jnp.int32)
        en = None
        for s in range(n_seg):
            st = plsc.load_gather(cpsall_v, [base]) if s == 0 else en
            en = plsc.load_gather(cpsall_v, [base + (s + 1)])
            ln = en - st
            hidx = st + jnp.minimum(hi_vec, ln)
            lidx = st + jnp.minimum(lo_vec, ln)
            qh = plsc.load_gather(qrow_v, [hidx - 1])
            if s == 0:
                ql = plsc.load_gather(qrow_v, [jnp.maximum(lidx - 1, 0)])
                ql = jnp.where(lidx > 0, ql, 0.0)
            else:
                ql = plsc.load_gather(qrow_v, [lidx - 1])
            orow_v[pl.ds(s * _LANES, _LANES)] = (qh - ql) * inv_vec

    do_row(n0, q0_v, o0_v)
    st0 = pltpu.async_copy(o0_v, out_hbm.at[n0], sem)
    do_row(n1c, q1_v, o1_v)

    @pl.when(has2)
    def _():
        pltpu.async_copy(o1_v, out_hbm.at[n1c], sem).wait()

    st0.wait()


def kernel(x, x_mark, change_points, W_pos, b_pos, W_te, b_te):
    batch, ts_len, ts_dim = x.shape
    n_rows = batch * ts_dim
    n_seg = change_points.shape[1] - 1

    xmark_t = jnp.transpose(x_mark[0])
    pebuf_t = jnp.asarray(_pe_table_t(_D_MODEL, ts_len))
    cps_pad = jnp.pad(change_points,
                      ((0, 0), (0, _CPS_PAD - (n_seg + 1))))

    vmem = pl.BlockSpec(memory_space=pltpu.VMEM)
    smem = pl.BlockSpec(memory_space=pltpu.SMEM)
    q, ei, ef = pl.pallas_call(
        _tc_body,
        out_shape=[
            jax.ShapeDtypeStruct((n_rows, ts_len), jnp.float32),
            jax.ShapeDtypeStruct((128,), jnp.int32),
            jax.ShapeDtypeStruct((128,), jnp.float32),
        ],
        in_specs=[vmem, vmem, vmem, vmem, vmem, smem, smem, vmem],
    )(x, pebuf_t, W_pos, xmark_t, W_te, b_pos, b_te, change_points)

    sc_fn = pl.kernel(
        functools.partial(_sc_body, n_rows, n_seg),
        out_type=jax.ShapeDtypeStruct((n_rows, n_seg * _N_BINS), jnp.float32),
        mesh=plsc.VectorSubcoreMesh(core_axis_name="c", subcore_axis_name="s"),
        compiler_params=pltpu.CompilerParams(needs_layout_passes=False),
        scratch_types=[
            pltpu.VMEM((n_rows * _CPS_PAD,), jnp.int32),
            pltpu.VMEM((8 * _N_BINS,), jnp.int32),
            pltpu.VMEM((8 * _N_BINS,), jnp.float32),
            pltpu.VMEM((ts_len,), jnp.float32),
            pltpu.VMEM((ts_len,), jnp.float32),
            pltpu.VMEM((n_seg * _N_BINS,), jnp.float32),
            pltpu.VMEM((n_seg * _N_BINS,), jnp.float32),
            pltpu.SemaphoreType.DMA,
        ],
    )
    out = sc_fn(q, cps_pad.reshape(-1), ei, ef)
    return out.reshape(batch, ts_dim, n_seg, _N_BINS)


# speedup vs baseline: 1.0380x; 1.0380x over previous
import functools
import math

import jax
import jax.numpy as jnp
import numpy as np
from jax import lax
from jax.experimental import pallas as pl
from jax.experimental.pallas import tpu as pltpu
from jax.experimental.pallas import tpu_sc as plsc

_D_MODEL = 128
_N_BINS = 16
_LANES = 16
_N_WORKERS = 32
_CPS_PAD = 32


def _pe_table(d_model, length):
    position = np.arange(length, dtype=np.float32)[:, None]
    div_term = np.exp(np.arange(0, d_model, 2, dtype=np.float32)
                      * -(math.log(10000.0) / d_model))
    pe = np.zeros((length, d_model), dtype=np.float32)
    pe[:, 0::2] = np.sin(position * div_term)
    pe[:, 1::2] = np.cos(position * div_term)
    return pe


def _tc_body(xcols_ref, pebuf_ref, wpos_ref, xmark_ref, wte_ref,
             bpos_ref, bte_ref, cps_ref, q_ref, ei_ref, ef_ref, cps2_ref):
    pe_col = jnp.dot(pebuf_ref[:], wpos_ref[:],
                     preferred_element_type=jnp.float32)
    te_col = jnp.dot(xmark_ref[:], wte_ref[:],
                     preferred_element_type=jnp.float32)
    bias = bpos_ref[0] + bte_ref[0]
    pete = jnp.transpose(pe_col + te_col, (1, 0))
    u = xcols_ref[:] + pete + bias
    length = u.shape[1]
    k = 1
    while k < length:
        shifted = jnp.concatenate(
            [jnp.zeros(u.shape[:-1] + (k,), u.dtype), u[:, :length - k]],
            axis=1)
        u = u + shifted
        k *= 2
    q_ref[:] = u
    cps = cps_ref[:]
    lens = cps[:, 1:] - cps[:, :-1]
    m = jnp.max(lens)
    i = lax.broadcasted_iota(jnp.int32, (1, _N_BINS), 1)
    lo = jnp.right_shift(i * m, 4)
    hi = jnp.right_shift((i + 1) * m + 15, 4)
    ei_ref[:] = jnp.reshape(
        jnp.concatenate(
            [lo, hi, jnp.zeros((1, 128 - 2 * _N_BINS), jnp.int32)], axis=1),
        (128,))
    inv = 1.0 / (hi - lo).astype(jnp.float32)
    ef_ref[:] = jnp.reshape(
        jnp.concatenate(
            [inv, jnp.zeros((1, 128 - _N_BINS), jnp.float32)], axis=1),
        (128,))
    cps2_ref[:] = jnp.concatenate(
        [cps, jnp.zeros((cps.shape[0], _CPS_PAD - cps.shape[1]), jnp.int32)],
        axis=1)


def _sc_body(n_rows, n_seg, q_hbm, cps_hbm, ei_hbm, ef_hbm, out_hbm,
             cp0_v, cp1_v, ei_v, ef_v, q0_v, q1_v, o0_v, o1_v, sem):
    wid = lax.axis_index("c") * 16 + lax.axis_index("s")
    n0 = wid
    n1 = wid + _N_WORKERS
    has2 = n1 < n_rows
    n1c = jnp.where(has2, n1, 0)
    copies = [
        pltpu.async_copy(cps_hbm.at[n0], cp0_v, sem),
        pltpu.async_copy(cps_hbm.at[n1c], cp1_v, sem),
        pltpu.async_copy(ei_hbm, ei_v, sem),
        pltpu.async_copy(ef_hbm, ef_v, sem),
        pltpu.async_copy(q_hbm.at[n0], q0_v, sem),
        pltpu.async_copy(q_hbm.at[n1c], q1_v, sem),
    ]
    for c in copies:
        c.wait()
    lo_vec = ei_v[pl.ds(0, _LANES)]
    hi_vec = ei_v[pl.ds(_LANES, _LANES)]
    inv_vec = ef_v[pl.ds(0, _LANES)]

    def do_row(cp_v, qrow_v, orow_v):
        en = None
        for s in range(n_seg):
            st = (plsc.load_gather(cp_v, [jnp.zeros((_LANES,), jnp.int32)])
                  if s == 0 else en)
            en = plsc.load_gather(cp_v, [jnp.full((_LANES,), s + 1, jnp.int32)])
            ln = en - st
            hidx = st + jnp.minimum(hi_vec, ln)
            lidx = st + jnp.minimum(lo_vec, ln)
            qh = plsc.load_gather(qrow_v, [hidx - 1])
            if s == 0:
                ql = plsc.load_gather(qrow_v, [jnp.maximum(lidx - 1, 0)])
                ql = jnp.where(lidx > 0, ql, 0.0)
            else:
                ql = plsc.load_gather(qrow_v, [lidx - 1])
            orow_v[pl.ds(s * _LANES, _LANES)] = (qh - ql) * inv_vec

    do_row(cp0_v, q0_v, o0_v)
    st0 = pltpu.async_copy(o0_v, out_hbm.at[n0], sem)
    do_row(cp1_v, q1_v, o1_v)

    @pl.when(has2)
    def _():
        pltpu.async_copy(o1_v, out_hbm.at[n1c], sem).wait()

    st0.wait()


def kernel(x, x_mark, change_points, W_pos, b_pos, W_te, b_te):
    batch, ts_len, ts_dim = x.shape
    n_rows = batch * ts_dim
    n_seg = change_points.shape[1] - 1

    xcols = jnp.transpose(x, (0, 2, 1)).reshape(n_rows, ts_len)
    pebuf = jnp.asarray(_pe_table(_D_MODEL, ts_len))

    vmem = pl.BlockSpec(memory_space=pltpu.VMEM)
    smem = pl.BlockSpec(memory_space=pltpu.SMEM)
    q, ei, ef, cps2 = pl.pallas_call(
        _tc_body,
        out_shape=[
            jax.ShapeDtypeStruct((n_rows, ts_len), jnp.float32),
            jax.ShapeDtypeStruct((128,), jnp.int32),
            jax.ShapeDtypeStruct((128,), jnp.float32),
            jax.ShapeDtypeStruct((n_rows, _CPS_PAD), jnp.int32),
        ],
        in_specs=[vmem, vmem, vmem, vmem, vmem, smem, smem, vmem],
    )(xcols, pebuf, W_pos, x_mark[0], W_te, b_pos, b_te, change_points)

    sc_fn = pl.kernel(
        functools.partial(_sc_body, n_rows, n_seg),
        out_type=jax.ShapeDtypeStruct((n_rows, n_seg * _N_BINS), jnp.float32),
        mesh=plsc.VectorSubcoreMesh(core_axis_name="c", subcore_axis_name="s"),
        compiler_params=pltpu.CompilerParams(needs_layout_passes=False),
        scratch_types=[
            pltpu.VMEM((_CPS_PAD,), jnp.int32),
            pltpu.VMEM((_CPS_PAD,), jnp.int32),
            pltpu.VMEM((8 * _N_BINS,), jnp.int32),
            pltpu.VMEM((8 * _N_BINS,), jnp.float32),
            pltpu.VMEM((ts_len,), jnp.float32),
            pltpu.VMEM((ts_len,), jnp.float32),
            pltpu.VMEM((n_seg * _N_BINS,), jnp.float32),
            pltpu.VMEM((n_seg * _N_BINS,), jnp.float32),
            pltpu.SemaphoreType.DMA,
        ],
    )
    out = sc_fn(q, cps2, ei, ef)
    return out.reshape(batch, ts_dim, n_seg, _N_BINS)


# speedup vs baseline: 1.0797x; 1.0401x over previous
import functools
import math

import jax
import jax.numpy as jnp
import numpy as np
from jax import lax
from jax.experimental import pallas as pl
from jax.experimental.pallas import tpu as pltpu
from jax.experimental.pallas import tpu_sc as plsc

_D_MODEL = 128
_N_BINS = 16
_LANES = 16
_N_WORKERS = 32
_CPS_PAD = 32


def _pe_table(d_model, length):
    position = np.arange(length, dtype=np.float32)[:, None]
    div_term = np.exp(np.arange(0, d_model, 2, dtype=np.float32)
                      * -(math.log(10000.0) / d_model))
    pe = np.zeros((length, d_model), dtype=np.float32)
    pe[:, 0::2] = np.sin(position * div_term)
    pe[:, 1::2] = np.cos(position * div_term)
    return pe


def _tc_body(xcols_ref, pebuf_ref, wpos_ref, xmark_ref, wte_ref,
             bpos_ref, bte_ref, cps_ref, q_ref, ei_ref, ef_ref):
    pe_col = jnp.dot(pebuf_ref[:], wpos_ref[:],
                     preferred_element_type=jnp.float32)
    te_col = jnp.dot(xmark_ref[:], wte_ref[:],
                     preferred_element_type=jnp.float32)
    bias = bpos_ref[0] + bte_ref[0]
    pete = jnp.transpose(pe_col + te_col, (1, 0))
    u = xcols_ref[:] + pete + bias
    length = u.shape[1]
    k = 1
    while k < length:
        shifted = jnp.concatenate(
            [jnp.zeros(u.shape[:-1] + (k,), u.dtype), u[:, :length - k]],
            axis=1)
        u = u + shifted
        k *= 2
    q_ref[:] = u
    cps = cps_ref[:]
    lens = cps[:, 1:] - cps[:, :-1]
    m = jnp.max(lens)
    i = lax.broadcasted_iota(jnp.int32, (1, _N_BINS), 1)
    lo = jnp.right_shift(i * m, 4)
    hi = jnp.right_shift((i + 1) * m + 15, 4)
    ei_ref[:] = jnp.reshape(
        jnp.concatenate(
            [lo, hi, jnp.zeros((1, 128 - 2 * _N_BINS), jnp.int32)], axis=1),
        (128,))
    inv = 1.0 / (hi - lo).astype(jnp.float32)
    ef_ref[:] = jnp.reshape(
        jnp.concatenate(
            [inv, jnp.zeros((1, 128 - _N_BINS), jnp.float32)], axis=1),
        (128,))


def _sc_body(n_rows, n_seg, q_hbm, cps_hbm, ei_hbm, ef_hbm, out_hbm,
             cpsall_v, ei_v, ef_v, q0_v, q1_v, o0_v, o1_v, sem):
    wid = lax.axis_index("c") * 16 + lax.axis_index("s")
    n0 = wid
    n1 = wid + _N_WORKERS
    has2 = n1 < n_rows
    n1c = jnp.where(has2, n1, 0)
    copies = [
        pltpu.async_copy(cps_hbm, cpsall_v, sem),
        pltpu.async_copy(ei_hbm, ei_v, sem),
        pltpu.async_copy(ef_hbm, ef_v, sem),
        pltpu.async_copy(q_hbm.at[n0], q0_v, sem),
        pltpu.async_copy(q_hbm.at[n1c], q1_v, sem),
    ]
    for c in copies:
        c.wait()
    lo_vec = ei_v[pl.ds(0, _LANES)]
    hi_vec = ei_v[pl.ds(_LANES, _LANES)]
    inv_vec = ef_v[pl.ds(0, _LANES)]

    def do_row(n, qrow_v, orow_v):
        base = jnp.full((_LANES,), n * _CPS_PAD, jnp.int32)
        en = None
        for s in range(n_seg):
            st = plsc.load_gather(cpsall_v, [base]) if s == 0 else en
            en = plsc.load_gather(cpsall_v, [base + (s + 1)])
            ln = en - st
            hidx = st + jnp.minimum(hi_vec, ln)
            lidx = st + jnp.minimum(lo_vec, ln)
            qh = plsc.load_gather(qrow_v, [hidx - 1])
            if s == 0:
                ql = plsc.load_gather(qrow_v, [jnp.maximum(lidx - 1, 0)])
                ql = jnp.where(lidx > 0, ql, 0.0)
            else:
                ql = plsc.load_gather(qrow_v, [lidx - 1])
            orow_v[pl.ds(s * _LANES, _LANES)] = (qh - ql) * inv_vec

    do_row(n0, q0_v, o0_v)
    st0 = pltpu.async_copy(o0_v, out_hbm.at[n0], sem)
    do_row(n1c, q1_v, o1_v)

    @pl.when(has2)
    def _():
        pltpu.async_copy(o1_v, out_hbm.at[n1c], sem).wait()

    st0.wait()


def kernel(x, x_mark, change_points, W_pos, b_pos, W_te, b_te):
    batch, ts_len, ts_dim = x.shape
    n_rows = batch * ts_dim
    n_seg = change_points.shape[1] - 1

    xcols = jnp.transpose(x, (0, 2, 1)).reshape(n_rows, ts_len)
    pebuf = jnp.asarray(_pe_table(_D_MODEL, ts_len))

    vmem = pl.BlockSpec(memory_space=pltpu.VMEM)
    smem = pl.BlockSpec(memory_space=pltpu.SMEM)
    cps_pad = jnp.pad(change_points,
                      ((0, 0), (0, _CPS_PAD - (n_seg + 1))))
    q, ei, ef = pl.pallas_call(
        _tc_body,
        out_shape=[
            jax.ShapeDtypeStruct((n_rows, ts_len), jnp.float32),
            jax.ShapeDtypeStruct((128,), jnp.int32),
            jax.ShapeDtypeStruct((128,), jnp.float32),
        ],
        in_specs=[vmem, vmem, vmem, vmem, vmem, smem, smem, vmem],
    )(xcols, pebuf, W_pos, x_mark[0], W_te, b_pos, b_te, change_points)

    sc_fn = pl.kernel(
        functools.partial(_sc_body, n_rows, n_seg),
        out_type=jax.ShapeDtypeStruct((n_rows, n_seg * _N_BINS), jnp.float32),
        mesh=plsc.VectorSubcoreMesh(core_axis_name="c", subcore_axis_name="s"),
        compiler_params=pltpu.CompilerParams(needs_layout_passes=False),
        scratch_types=[
            pltpu.VMEM((n_rows * _CPS_PAD,), jnp.int32),
            pltpu.VMEM((8 * _N_BINS,), jnp.int32),
            pltpu.VMEM((8 * _N_BINS,), jnp.float32),
            pltpu.VMEM((ts_len,), jnp.float32),
            pltpu.VMEM((ts_len,), jnp.float32),
            pltpu.VMEM((n_seg * _N_BINS,), jnp.float32),
            pltpu.VMEM((n_seg * _N_BINS,), jnp.float32),
            pltpu.SemaphoreType.DMA,
        ],
    )
    out = sc_fn(q, cps_pad.reshape(-1), ei, ef)
    return out.reshape(batch, ts_dim, n_seg, _N_BINS)
